# Initial kernel scaffold; baseline (speedup 1.0000x reference)
import jax
import jax.numpy as jnp
from jax.experimental import pallas as pl


def kernel(rois, probs, deltas):
    raise NotImplementedError("write your pallas kernel here")


# speedup vs baseline: 28.8136x; 28.8136x over previous
import jax
import jax.numpy as jnp
from jax import lax
from jax.experimental import pallas as pl
from jax.experimental.pallas import tpu as pltpu

_N = 5000
_NC = 81
_NP = 5120
_CH = 128
_NCHUNK = _NP // _CH
_IMG = 1024.0
_OFFSET = 2.0 * _IMG + 1.0
_MINCONF = 0.05
_NMS_T = 0.3
_MAXDET = 100


def _stage_a(rois_ref, probs_ref, deltas_ref,
             ref_out, nms_out, cid_out, score_out, mscore_out):
    probs = probs_ref[...]
    m = jnp.max(probs, axis=1, keepdims=True)
    lane_c = lax.broadcasted_iota(jnp.int32, probs.shape, 1)
    cid = jnp.min(jnp.where(probs == m, lane_c, _NC), axis=1, keepdims=True)

    d = deltas_ref[...]
    lane = lax.broadcasted_iota(jnp.int32, d.shape, 1)
    colc = lane // 4
    comp = lane - colc * 4
    seld = jnp.where(colc == cid, d, 0.0)
    dy = jnp.sum(jnp.where(comp == 0, seld, 0.0), axis=1, keepdims=True) * 0.1
    dx = jnp.sum(jnp.where(comp == 1, seld, 0.0), axis=1, keepdims=True) * 0.1
    dh = jnp.sum(jnp.where(comp == 2, seld, 0.0), axis=1, keepdims=True) * 0.2
    dw = jnp.sum(jnp.where(comp == 3, seld, 0.0), axis=1, keepdims=True) * 0.2

    y1 = rois_ref[:, 0:1]
    x1 = rois_ref[:, 1:2]
    y2 = rois_ref[:, 2:3]
    x2 = rois_ref[:, 3:4]
    h = y2 - y1
    w = x2 - x1
    cy = y1 + 0.5 * h + dy * h
    cx = x1 + 0.5 * w + dx * w
    h = h * jnp.exp(dh)
    w = w * jnp.exp(dw)
    ny1 = cy - 0.5 * h
    nx1 = cx - 0.5 * w
    ny2 = ny1 + h
    nx2 = nx1 + w
    ry1 = jnp.round(jnp.clip(ny1 * _IMG, 0.0, _IMG))
    rx1 = jnp.round(jnp.clip(nx1 * _IMG, 0.0, _IMG))
    ry2 = jnp.round(jnp.clip(ny2 * _IMG, 0.0, _IMG))
    rx2 = jnp.round(jnp.clip(nx2 * _IMG, 0.0, _IMG))
    refined = jnp.concatenate([ry1, rx1, ry2, rx2], axis=1)

    cidf = cid.astype(jnp.float32)
    ref_out[...] = refined
    nms_out[...] = refined + cidf * _OFFSET
    cid_out[...] = cidf
    score_out[...] = m
    keep = (cid > 0) & (m >= _MINCONF)
    mscore_out[...] = jnp.where(keep, m, -1.0)


def _stage_b(bcol_ref, scol_ref, brow_ref, srow_ref,
             refT_ref, cidT_ref, scoreT_ref, out_ref,
             trow_ref, tcol_ref, tnew_ref):
    f32 = jnp.float32
    srow = srow_ref[...]
    keep_row = srow >= 0.0
    idxr = lax.broadcasted_iota(jnp.int32, (1, _NP), 1)
    ry1 = brow_ref[0:1, :]
    rx1 = brow_ref[1:2, :]
    ry2 = brow_ref[2:3, :]
    rx2 = brow_ref[3:4, :]
    area_r = (ry2 - ry1) * (rx2 - rx1)

    trow_ref[...] = srow
    tcol_ref[...] = scol_ref[...]

    def iter_body(carry):
        it, _ = carry
        trow = trow_ref[...]

        def chunk_body(c, supp_row):
            i0 = c * _CH
            yc1 = bcol_ref[pl.ds(i0, _CH), 0:1]
            xc1 = bcol_ref[pl.ds(i0, _CH), 1:2]
            yc2 = bcol_ref[pl.ds(i0, _CH), 2:3]
            xc2 = bcol_ref[pl.ds(i0, _CH), 3:4]
            tc = tcol_ref[pl.ds(i0, _CH), :]
            sc = scol_ref[pl.ds(i0, _CH), :]
            idxc = i0 + lax.broadcasted_iota(jnp.int32, (_CH, 1), 0)
            area_c = (yc2 - yc1) * (xc2 - xc1)
            yy1 = jnp.maximum(yc1, ry1)
            xx1 = jnp.maximum(xc1, rx1)
            yy2 = jnp.minimum(yc2, ry2)
            xx2 = jnp.minimum(xc2, rx2)
            inter = jnp.maximum(yy2 - yy1, 0.0) * jnp.maximum(xx2 - xx1, 0.0)
            union = area_c + area_r - inter
            ov = inter / jnp.maximum(union, 1e-8) > _NMS_T
            prec1 = (tc > srow) | ((tc == srow) & (idxc < idxr))
            hit1 = jnp.where(ov & prec1, 1.0, 0.0)
            supp_row = jnp.maximum(supp_row, jnp.max(hit1, axis=0,
                                                     keepdims=True))
            prec2 = (trow > sc) | ((trow == sc) & (idxr < idxc))
            suppc = jnp.any(ov & prec2, axis=1, keepdims=True)
            tnew_ref[pl.ds(i0, _CH), :] = jnp.where(
                (sc >= 0.0) & ~suppc, sc, -1.0)
            return supp_row

        supp_row = lax.fori_loop(0, _NCHUNK, chunk_body,
                                 jnp.zeros((1, _NP), jnp.float32))
        tnew_row = jnp.where(keep_row & (supp_row == 0.0), srow, -1.0)
        changed = jnp.any(tnew_row != trow)
        trow_ref[...] = tnew_row
        tcol_ref[...] = tnew_ref[...]
        return (it + 1, changed)

    lax.while_loop(lambda c: c[1] & (c[0] < _NP + 2), iter_body,
                   (jnp.int32(0), jnp.bool_(True)))

    ref0 = refT_ref[0:1, :]
    ref1 = refT_ref[1:2, :]
    ref2 = refT_ref[2:3, :]
    ref3 = refT_ref[3:4, :]
    cidr = cidT_ref[...]
    scr = scoreT_ref[...]
    lane128 = lax.broadcasted_iota(jnp.int32, (1, 128), 1)

    def sel_body(k, carry):
        f, a0, a1, a2, a3, a4, a5 = carry
        mval = jnp.max(f)
        imin = jnp.min(jnp.where(f == mval, idxr, _NP))
        valid = jnp.where(mval > 0.0, f32(1.0), f32(0.0))
        onehot = idxr == imin

        def g(row):
            return jnp.sum(jnp.where(onehot, row, 0.0)) * valid

        put = lane128 == k
        a0 = jnp.where(put, g(ref0), a0)
        a1 = jnp.where(put, g(ref1), a1)
        a2 = jnp.where(put, g(ref2), a2)
        a3 = jnp.where(put, g(ref3), a3)
        a4 = jnp.where(put, g(cidr), a4)
        a5 = jnp.where(put, g(scr), a5)
        f = jnp.where(onehot, -2.0, f)
        return (f, a0, a1, a2, a3, a4, a5)

    z = jnp.zeros((1, 128), f32)
    f0 = trow_ref[...]
    _, a0, a1, a2, a3, a4, a5 = lax.fori_loop(
        0, _MAXDET, sel_body, (f0, z, z, z, z, z, z))
    out_ref[0:1, :] = a0
    out_ref[1:2, :] = a1
    out_ref[2:3, :] = a2
    out_ref[3:4, :] = a3
    out_ref[4:5, :] = a4
    out_ref[5:6, :] = a5
    out_ref[6:7, :] = z
    out_ref[7:8, :] = z


def kernel(rois, probs, deltas):
    f32 = jnp.float32
    deltas_r = deltas.reshape(_N, _NC * 4)
    refined, nmsb, cidf, score, mscore = pl.pallas_call(
        _stage_a,
        out_shape=[
            jax.ShapeDtypeStruct((_N, 4), f32),
            jax.ShapeDtypeStruct((_N, 4), f32),
            jax.ShapeDtypeStruct((_N, 1), f32),
            jax.ShapeDtypeStruct((_N, 1), f32),
            jax.ShapeDtypeStruct((_N, 1), f32),
        ],
    )(rois, probs, deltas_r)

    pad = _NP - _N
    nmsb_p = jnp.pad(nmsb, ((0, pad), (0, 0)))
    mscore_p = jnp.pad(mscore, ((0, pad), (0, 0)), constant_values=-1.0)
    refined_p = jnp.pad(refined, ((0, pad), (0, 0)))
    cid_p = jnp.pad(cidf, ((0, pad), (0, 0)))
    score_p = jnp.pad(score, ((0, pad), (0, 0)))

    out = pl.pallas_call(
        _stage_b,
        out_shape=jax.ShapeDtypeStruct((8, 128), f32),
        scratch_shapes=[
            pltpu.VMEM((1, _NP), f32),
            pltpu.VMEM((_NP, 1), f32),
            pltpu.VMEM((_NP, 1), f32),
        ],
    )(nmsb_p, mscore_p, nmsb_p.T, mscore_p.T,
      refined_p.T, cid_p.T, score_p.T)

    return out[:6, :].T[:_MAXDET, :]


# speedup vs baseline: 69.8560x; 2.4244x over previous
import jax
import jax.numpy as jnp
from jax import lax
from jax.experimental import pallas as pl
from jax.experimental.pallas import tpu as pltpu

_N = 5000
_NC = 81
_NP = 5120
_CH = 128
_NCHUNK = _NP // _CH
_IMG = 1024.0
_OFFSET = 2.0 * _IMG + 1.0
_MINCONF = 0.05
_NMS_T = 0.3
_MAXDET = 100


def _stage_a(rois_ref, probs_ref, deltas_ref,
             ref_out, nms_out, cid_out, score_out, mscore_out):
    probs = probs_ref[...]
    m = jnp.max(probs, axis=1, keepdims=True)
    lane_c = lax.broadcasted_iota(jnp.int32, probs.shape, 1)
    cid = jnp.min(jnp.where(probs == m, lane_c, _NC), axis=1, keepdims=True)

    d = deltas_ref[...]
    lane = lax.broadcasted_iota(jnp.int32, d.shape, 1)
    colc = lane // 4
    comp = lane - colc * 4
    seld = jnp.where(colc == cid, d, 0.0)
    dy = jnp.sum(jnp.where(comp == 0, seld, 0.0), axis=1, keepdims=True) * 0.1
    dx = jnp.sum(jnp.where(comp == 1, seld, 0.0), axis=1, keepdims=True) * 0.1
    dh = jnp.sum(jnp.where(comp == 2, seld, 0.0), axis=1, keepdims=True) * 0.2
    dw = jnp.sum(jnp.where(comp == 3, seld, 0.0), axis=1, keepdims=True) * 0.2

    y1 = rois_ref[:, 0:1]
    x1 = rois_ref[:, 1:2]
    y2 = rois_ref[:, 2:3]
    x2 = rois_ref[:, 3:4]
    h = y2 - y1
    w = x2 - x1
    cy = y1 + 0.5 * h + dy * h
    cx = x1 + 0.5 * w + dx * w
    h = h * jnp.exp(dh)
    w = w * jnp.exp(dw)
    ny1 = cy - 0.5 * h
    nx1 = cx - 0.5 * w
    ny2 = ny1 + h
    nx2 = nx1 + w
    ry1 = jnp.round(jnp.clip(ny1 * _IMG, 0.0, _IMG))
    rx1 = jnp.round(jnp.clip(nx1 * _IMG, 0.0, _IMG))
    ry2 = jnp.round(jnp.clip(ny2 * _IMG, 0.0, _IMG))
    rx2 = jnp.round(jnp.clip(nx2 * _IMG, 0.0, _IMG))
    refined = jnp.concatenate([ry1, rx1, ry2, rx2], axis=1)

    cidf = cid.astype(jnp.float32)
    ref_out[...] = refined
    nms_out[...] = refined + cidf * _OFFSET
    cid_out[...] = cidf
    score_out[...] = m
    keep = (cid > 0) & (m >= _MINCONF)
    mscore_out[...] = jnp.where(keep, m, -1.0)


def _stage_b(bcol_ref, scol_ref, brow_ref, srow_ref, pack_ref, out_ref,
             m_ref, trow_ref, tcol_ref):
    f32 = jnp.float32
    srow = srow_ref[...]
    keep_row = srow >= 0.0
    idxr = lax.broadcasted_iota(jnp.int32, (1, _NP), 1)
    ry1 = brow_ref[0:1, :]
    rx1 = brow_ref[1:2, :]
    ry2 = brow_ref[2:3, :]
    rx2 = brow_ref[3:4, :]
    area_r = (ry2 - ry1) * (rx2 - rx1)
    eye = (lax.broadcasted_iota(jnp.int32, (_CH, _CH), 0)
           == lax.broadcasted_iota(jnp.int32, (_CH, _CH), 1))

    def build_body(c, _):
        i0 = c * _CH
        yc1 = bcol_ref[pl.ds(i0, _CH), 0:1]
        xc1 = bcol_ref[pl.ds(i0, _CH), 1:2]
        yc2 = bcol_ref[pl.ds(i0, _CH), 2:3]
        xc2 = bcol_ref[pl.ds(i0, _CH), 3:4]
        sc = scol_ref[pl.ds(i0, _CH), :]
        idxc = i0 + lax.broadcasted_iota(jnp.int32, (_CH, 1), 0)
        area_c = (yc2 - yc1) * (xc2 - xc1)
        yy1 = jnp.maximum(yc1, ry1)
        xx1 = jnp.maximum(xc1, rx1)
        yy2 = jnp.minimum(yc2, ry2)
        xx2 = jnp.minimum(xc2, rx2)
        inter = jnp.maximum(yy2 - yy1, 0.0) * jnp.maximum(xx2 - xx1, 0.0)
        union = area_c + area_r - inter
        ov = inter > _NMS_T * union
        prec = (sc > srow) | ((sc == srow) & (idxc < idxr))
        m_ref[pl.ds(i0, _CH), :] = jnp.where(ov & prec, 1, 0).astype(jnp.int8)
        return 0

    lax.fori_loop(0, _NCHUNK, build_body, 0)

    trow_ref[...] = srow
    tcol_ref[...] = scol_ref[...]

    def iter_body(carry):
        it, _ = carry
        trow = trow_ref[...]

        def chunk_body(c, supp_row):
            i0 = c * _CH
            mcf = m_ref[pl.ds(i0, _CH), :].astype(jnp.float32)
            alive_f = jnp.where(
                tcol_ref[pl.ds(i0, _CH), :] >= 0.0, 1.0, 0.0)
            hit = mcf * alive_f
            return jnp.maximum(supp_row,
                               jnp.max(hit, axis=0, keepdims=True))

        supp_row = lax.fori_loop(0, _NCHUNK, chunk_body,
                                 jnp.zeros((1, _NP), jnp.float32))
        tnew_row = jnp.where(keep_row & (supp_row == 0.0), srow, -1.0)
        changed = jnp.any(tnew_row != trow)
        trow_ref[...] = tnew_row

        def tr_body(c, _):
            i0 = c * _CH
            rsl = trow_ref[0:1, pl.ds(i0, _CH)]
            tcol_ref[pl.ds(i0, _CH), :] = jnp.sum(
                jnp.where(eye, rsl, 0.0), axis=1, keepdims=True)
            return 0

        lax.fori_loop(0, _NCHUNK, tr_body, 0)
        return (it + 1, changed)

    lax.while_loop(lambda c: c[1] & (c[0] < _NP + 2), iter_body,
                   (jnp.int32(0), jnp.bool_(True)))

    pack = pack_ref[...]
    lane128 = lax.broadcasted_iota(jnp.int32, (1, 128), 1)

    def sel_body(k, carry):
        f, acc = carry
        mval = jnp.max(f)
        imin = jnp.min(jnp.where(f == mval, idxr, _NP))
        valid = jnp.where(mval > 0.0, f32(1.0), f32(0.0))
        onehot = idxr == imin
        ohf = jnp.where(onehot, 1.0, 0.0)
        vals = jnp.sum(pack * ohf, axis=1,
                       keepdims=True) * valid
        acc = jnp.where(lane128 == k, vals, acc)
        f = jnp.where(onehot, -2.0, f)
        return (f, acc)

    f0 = trow_ref[...]
    _, acc = lax.fori_loop(0, _MAXDET, sel_body,
                           (f0, jnp.zeros((8, 128), f32)))
    out_ref[...] = acc


def kernel(rois, probs, deltas):
    f32 = jnp.float32
    deltas_r = deltas.reshape(_N, _NC * 4)
    refined, nmsb, cidf, score, mscore = pl.pallas_call(
        _stage_a,
        out_shape=[
            jax.ShapeDtypeStruct((_N, 4), f32),
            jax.ShapeDtypeStruct((_N, 4), f32),
            jax.ShapeDtypeStruct((_N, 1), f32),
            jax.ShapeDtypeStruct((_N, 1), f32),
            jax.ShapeDtypeStruct((_N, 1), f32),
        ],
    )(rois, probs, deltas_r)

    pad = _NP - _N
    nmsb_p = jnp.pad(nmsb, ((0, pad), (0, 0)))
    mscore_p = jnp.pad(mscore, ((0, pad), (0, 0)), constant_values=-1.0)
    refined_p = jnp.pad(refined, ((0, pad), (0, 0)))
    cid_p = jnp.pad(cidf, ((0, pad), (0, 0)))
    score_p = jnp.pad(score, ((0, pad), (0, 0)))
    packed = jnp.concatenate(
        [refined_p.T, cid_p.T, score_p.T, jnp.zeros((2, _NP), f32)], axis=0)

    out = pl.pallas_call(
        _stage_b,
        out_shape=jax.ShapeDtypeStruct((8, 128), f32),
        scratch_shapes=[
            pltpu.VMEM((_NP, _NP), jnp.int8),
            pltpu.VMEM((1, _NP), f32),
            pltpu.VMEM((_NP, 1), f32),
        ],
    )(nmsb_p, mscore_p, nmsb_p.T, mscore_p.T, packed)

    return out[:6, :].T[:_MAXDET, :]


# speedup vs baseline: 90.9878x; 1.3025x over previous
import jax
import jax.numpy as jnp
from jax import lax
from jax.experimental import pallas as pl
from jax.experimental.pallas import tpu as pltpu

_N = 5000
_NC = 81
_NP = 5120
_CH = 128
_NCHUNK = _NP // _CH
_IMG = 1024.0
_OFFSET = 2.0 * _IMG + 1.0
_MINCONF = 0.05
_NMS_T = 0.3
_MAXDET = 100


def _stage_a(rois_ref, probs_ref, deltas_ref,
             ref_out, nms_out, cid_out, score_out, mscore_out):
    probs = probs_ref[...]
    m = jnp.max(probs, axis=1, keepdims=True)
    lane_c = lax.broadcasted_iota(jnp.int32, probs.shape, 1)
    cid = jnp.min(jnp.where(probs == m, lane_c, _NC), axis=1, keepdims=True)

    d = deltas_ref[...]
    lane = lax.broadcasted_iota(jnp.int32, d.shape, 1)
    colc = lane // 4
    comp = lane - colc * 4
    seld = jnp.where(colc == cid, d, 0.0)
    dy = jnp.sum(jnp.where(comp == 0, seld, 0.0), axis=1, keepdims=True) * 0.1
    dx = jnp.sum(jnp.where(comp == 1, seld, 0.0), axis=1, keepdims=True) * 0.1
    dh = jnp.sum(jnp.where(comp == 2, seld, 0.0), axis=1, keepdims=True) * 0.2
    dw = jnp.sum(jnp.where(comp == 3, seld, 0.0), axis=1, keepdims=True) * 0.2

    y1 = rois_ref[:, 0:1]
    x1 = rois_ref[:, 1:2]
    y2 = rois_ref[:, 2:3]
    x2 = rois_ref[:, 3:4]
    h = y2 - y1
    w = x2 - x1
    cy = y1 + 0.5 * h + dy * h
    cx = x1 + 0.5 * w + dx * w
    h = h * jnp.exp(dh)
    w = w * jnp.exp(dw)
    ny1 = cy - 0.5 * h
    nx1 = cx - 0.5 * w
    ny2 = ny1 + h
    nx2 = nx1 + w
    ry1 = jnp.round(jnp.clip(ny1 * _IMG, 0.0, _IMG))
    rx1 = jnp.round(jnp.clip(nx1 * _IMG, 0.0, _IMG))
    ry2 = jnp.round(jnp.clip(ny2 * _IMG, 0.0, _IMG))
    rx2 = jnp.round(jnp.clip(nx2 * _IMG, 0.0, _IMG))
    refined = jnp.concatenate([ry1, rx1, ry2, rx2], axis=1)

    cidf = cid.astype(jnp.float32)
    ref_out[...] = refined
    nms_out[...] = refined + cidf * _OFFSET
    cid_out[...] = cidf
    score_out[...] = m
    keep = (cid > 0) & (m >= _MINCONF)
    mscore_out[...] = jnp.where(keep, m, -1.0)


def _stage_b(bcol_ref, scol_ref, brow_ref, srow_ref, pack_ref, out_ref,
             m_ref, trow_ref, arow_ref):
    f32 = jnp.float32
    srow = srow_ref[...]
    keep_row = srow >= 0.0
    idxr = lax.broadcasted_iota(jnp.int32, (1, _NP), 1)
    ry1 = brow_ref[0:1, :]
    rx1 = brow_ref[1:2, :]
    ry2 = brow_ref[2:3, :]
    rx2 = brow_ref[3:4, :]
    area_r = (ry2 - ry1) * (rx2 - rx1)

    def build_body(c, _):
        i0 = c * _CH
        yc1 = bcol_ref[pl.ds(i0, _CH), 0:1]
        xc1 = bcol_ref[pl.ds(i0, _CH), 1:2]
        yc2 = bcol_ref[pl.ds(i0, _CH), 2:3]
        xc2 = bcol_ref[pl.ds(i0, _CH), 3:4]
        sc = scol_ref[pl.ds(i0, _CH), :]
        idxc = i0 + lax.broadcasted_iota(jnp.int32, (_CH, 1), 0)
        area_c = (yc2 - yc1) * (xc2 - xc1)
        yy1 = jnp.maximum(yc1, ry1)
        xx1 = jnp.maximum(xc1, rx1)
        yy2 = jnp.minimum(yc2, ry2)
        xx2 = jnp.minimum(xc2, rx2)
        inter = jnp.maximum(yy2 - yy1, 0.0) * jnp.maximum(xx2 - xx1, 0.0)
        union = area_c + area_r - inter
        ov = inter > _NMS_T * union
        prec = (sc > srow) | ((sc == srow) & (idxc < idxr))
        m_ref[pl.ds(i0, _CH), :] = jnp.where(ov & prec, 1, 0).astype(jnp.int8)
        return 0

    lax.fori_loop(0, _NCHUNK, build_body, 0)

    trow_ref[...] = srow
    arow_ref[...] = jnp.where(srow >= 0.0, 1, 0).astype(jnp.int8)

    def iter_body(carry):
        it, _ = carry
        trow = trow_ref[...]
        supp = lax.dot_general(
            arow_ref[...], m_ref[...], (((1,), (0,)), ((), ())),
            preferred_element_type=jnp.int32)
        tnew_row = jnp.where(keep_row & (supp == 0), srow, -1.0)
        changed = jnp.any(tnew_row != trow)
        trow_ref[...] = tnew_row
        arow_ref[...] = jnp.where(tnew_row >= 0.0, 1, 0).astype(jnp.int8)
        return (it + 1, changed)

    lax.while_loop(lambda c: c[1] & (c[0] < _NP + 2), iter_body,
                   (jnp.int32(0), jnp.bool_(True)))

    pack = pack_ref[...]
    lane128 = lax.broadcasted_iota(jnp.int32, (1, 128), 1)

    def sel_body(k, carry):
        f, acc = carry
        mval = jnp.max(f)
        imin = jnp.min(jnp.where(f == mval, idxr, _NP))
        valid = jnp.where(mval > 0.0, f32(1.0), f32(0.0))
        onehot = idxr == imin
        ohf = jnp.where(onehot, 1.0, 0.0)
        vals = jnp.sum(pack * ohf, axis=1,
                       keepdims=True) * valid
        acc = jnp.where(lane128 == k, vals, acc)
        f = jnp.where(onehot, -2.0, f)
        return (f, acc)

    f0 = trow_ref[...]
    _, acc = lax.fori_loop(0, _MAXDET, sel_body,
                           (f0, jnp.zeros((8, 128), f32)))
    out_ref[...] = acc


def kernel(rois, probs, deltas):
    f32 = jnp.float32
    deltas_r = deltas.reshape(_N, _NC * 4)
    refined, nmsb, cidf, score, mscore = pl.pallas_call(
        _stage_a,
        out_shape=[
            jax.ShapeDtypeStruct((_N, 4), f32),
            jax.ShapeDtypeStruct((_N, 4), f32),
            jax.ShapeDtypeStruct((_N, 1), f32),
            jax.ShapeDtypeStruct((_N, 1), f32),
            jax.ShapeDtypeStruct((_N, 1), f32),
        ],
    )(rois, probs, deltas_r)

    pad = _NP - _N
    nmsb_p = jnp.pad(nmsb, ((0, pad), (0, 0)))
    mscore_p = jnp.pad(mscore, ((0, pad), (0, 0)), constant_values=-1.0)
    refined_p = jnp.pad(refined, ((0, pad), (0, 0)))
    cid_p = jnp.pad(cidf, ((0, pad), (0, 0)))
    score_p = jnp.pad(score, ((0, pad), (0, 0)))
    packed = jnp.concatenate(
        [refined_p.T, cid_p.T, score_p.T, jnp.zeros((2, _NP), f32)], axis=0)

    out = pl.pallas_call(
        _stage_b,
        out_shape=jax.ShapeDtypeStruct((8, 128), f32),
        scratch_shapes=[
            pltpu.VMEM((_NP, _NP), jnp.int8),
            pltpu.VMEM((1, _NP), f32),
            pltpu.VMEM((1, _NP), jnp.int8),
        ],
    )(nmsb_p, mscore_p, nmsb_p.T, mscore_p.T, packed)

    return out[:6, :].T[:_MAXDET, :]


# speedup vs baseline: 102.9037x; 1.1310x over previous
import jax
import jax.numpy as jnp
from jax import lax
from jax.experimental import pallas as pl
from jax.experimental.pallas import tpu as pltpu

_N = 5000
_NC = 81
_NP = 5120
_CH = 128
_NCHUNK = _NP // _CH
_IMG = 1024.0
_OFFSET = 2.0 * _IMG + 1.0
_MINCONF = 0.05
_NMS_T = 0.3
_MAXDET = 100


def _stage_a(rois_ref, probs_ref, deltas_ref,
             ref_out, nms_out, cid_out, score_out, mscore_out):
    probs = probs_ref[...]
    m = jnp.max(probs, axis=1, keepdims=True)
    lane_c = lax.broadcasted_iota(jnp.int32, probs.shape, 1)
    cid = jnp.min(jnp.where(probs == m, lane_c, _NC), axis=1, keepdims=True)

    d = deltas_ref[...]
    lane = lax.broadcasted_iota(jnp.int32, d.shape, 1)
    colc = lane // 4
    comp = lane - colc * 4
    seld = jnp.where(colc == cid, d, 0.0)
    dy = jnp.sum(jnp.where(comp == 0, seld, 0.0), axis=1, keepdims=True) * 0.1
    dx = jnp.sum(jnp.where(comp == 1, seld, 0.0), axis=1, keepdims=True) * 0.1
    dh = jnp.sum(jnp.where(comp == 2, seld, 0.0), axis=1, keepdims=True) * 0.2
    dw = jnp.sum(jnp.where(comp == 3, seld, 0.0), axis=1, keepdims=True) * 0.2

    y1 = rois_ref[:, 0:1]
    x1 = rois_ref[:, 1:2]
    y2 = rois_ref[:, 2:3]
    x2 = rois_ref[:, 3:4]
    h = y2 - y1
    w = x2 - x1
    cy = y1 + 0.5 * h + dy * h
    cx = x1 + 0.5 * w + dx * w
    h = h * jnp.exp(dh)
    w = w * jnp.exp(dw)
    ny1 = cy - 0.5 * h
    nx1 = cx - 0.5 * w
    ny2 = ny1 + h
    nx2 = nx1 + w
    ry1 = jnp.round(jnp.clip(ny1 * _IMG, 0.0, _IMG))
    rx1 = jnp.round(jnp.clip(nx1 * _IMG, 0.0, _IMG))
    ry2 = jnp.round(jnp.clip(ny2 * _IMG, 0.0, _IMG))
    rx2 = jnp.round(jnp.clip(nx2 * _IMG, 0.0, _IMG))
    refined = jnp.concatenate([ry1, rx1, ry2, rx2], axis=1)

    cidf = cid.astype(jnp.float32)
    ref_out[...] = refined
    nms_out[...] = refined + cidf * _OFFSET
    cid_out[...] = cidf
    score_out[...] = m
    keep = (cid > 0) & (m >= _MINCONF)
    mscore_out[...] = jnp.where(keep, m, -1.0)


def _stage_b(bcol_ref, scol_ref, brow_ref, srow_ref, pack_ref, out_ref,
             m_ref, p8_ref, trow_ref, arow_ref, rcol_ref):
    f32 = jnp.float32
    srow = srow_ref[...]
    keep_row = srow >= 0.0
    idxr = lax.broadcasted_iota(jnp.int32, (1, _NP), 1)
    ry1 = brow_ref[0:1, :]
    rx1 = brow_ref[1:2, :]
    ry2 = brow_ref[2:3, :]
    rx2 = brow_ref[3:4, :]
    area_r = (ry2 - ry1) * (rx2 - rx1)

    def build_body(c, _):
        i0 = c * _CH
        yc1 = bcol_ref[pl.ds(i0, _CH), 0:1]
        xc1 = bcol_ref[pl.ds(i0, _CH), 1:2]
        yc2 = bcol_ref[pl.ds(i0, _CH), 2:3]
        xc2 = bcol_ref[pl.ds(i0, _CH), 3:4]
        sc = scol_ref[pl.ds(i0, _CH), :]
        idxc = i0 + lax.broadcasted_iota(jnp.int32, (_CH, 1), 0)
        area_c = (yc2 - yc1) * (xc2 - xc1)
        yy1 = jnp.maximum(yc1, ry1)
        xx1 = jnp.maximum(xc1, rx1)
        yy2 = jnp.minimum(yc2, ry2)
        xx2 = jnp.minimum(xc2, rx2)
        inter = jnp.maximum(yy2 - yy1, 0.0) * jnp.maximum(xx2 - xx1, 0.0)
        union = area_c + area_r - inter
        ov = inter > _NMS_T * union
        prec = (sc > srow) | ((sc == srow) & (idxc < idxr))
        m_ref[pl.ds(i0, _CH), :] = jnp.where(ov & prec, 1, 0).astype(jnp.int8)
        p8_ref[pl.ds(i0, _CH), :] = jnp.where(prec, 1, 0).astype(jnp.int8)
        return 0

    lax.fori_loop(0, _NCHUNK, build_body, 0)

    trow_ref[...] = srow
    arow_ref[...] = jnp.where(srow >= 0.0, 1, 0).astype(jnp.int8)

    def iter_body(carry):
        it, _ = carry
        trow = trow_ref[...]
        supp = lax.dot_general(
            arow_ref[...], m_ref[...], (((1,), (0,)), ((), ())),
            preferred_element_type=jnp.int32)
        tnew_row = jnp.where(keep_row & (supp == 0), srow, -1.0)
        changed = jnp.any(tnew_row != trow)
        trow_ref[...] = tnew_row
        arow_ref[...] = jnp.where(tnew_row >= 0.0, 1, 0).astype(jnp.int8)
        return (it + 1, changed)

    lax.while_loop(lambda c: c[1] & (c[0] < _NP + 2), iter_body,
                   (jnp.int32(0), jnp.bool_(True)))

    f = trow_ref[...]
    pos8 = jnp.where(f > 0.0, 1, 0).astype(jnp.int8)
    rank = lax.dot_general(
        pos8, p8_ref[...], (((1,), (0,)), ((), ())),
        preferred_element_type=jnp.int32)
    trow_ref[...] = jnp.where(f > 0.0, rank.astype(f32), f32(_NP))

    eye = (lax.broadcasted_iota(jnp.int32, (_CH, _CH), 0)
           == lax.broadcasted_iota(jnp.int32, (_CH, _CH), 1))

    def tr_body(c, _):
        i0 = c * _CH
        rsl = trow_ref[0:1, pl.ds(i0, _CH)]
        rcol_ref[pl.ds(i0, _CH), :] = jnp.sum(
            jnp.where(eye, rsl, 0.0), axis=1, keepdims=True)
        return 0

    lax.fori_loop(0, _NCHUNK, tr_body, 0)
    lanek = lax.broadcasted_iota(jnp.int32, (1, 128), 1).astype(f32)
    ohf = jnp.where(rcol_ref[...] == lanek, 1.0, 0.0)
    out_ref[...] = lax.dot_general(
        pack_ref[...], ohf, (((1,), (0,)), ((), ())),
        precision=lax.Precision.HIGHEST,
        preferred_element_type=f32)


def kernel(rois, probs, deltas):
    f32 = jnp.float32
    deltas_r = deltas.reshape(_N, _NC * 4)
    refined, nmsb, cidf, score, mscore = pl.pallas_call(
        _stage_a,
        out_shape=[
            jax.ShapeDtypeStruct((_N, 4), f32),
            jax.ShapeDtypeStruct((_N, 4), f32),
            jax.ShapeDtypeStruct((_N, 1), f32),
            jax.ShapeDtypeStruct((_N, 1), f32),
            jax.ShapeDtypeStruct((_N, 1), f32),
        ],
    )(rois, probs, deltas_r)

    pad = _NP - _N
    nmsb_p = jnp.pad(nmsb, ((0, pad), (0, 0)))
    mscore_p = jnp.pad(mscore, ((0, pad), (0, 0)), constant_values=-1.0)
    refined_p = jnp.pad(refined, ((0, pad), (0, 0)))
    cid_p = jnp.pad(cidf, ((0, pad), (0, 0)))
    score_p = jnp.pad(score, ((0, pad), (0, 0)))
    packed = jnp.concatenate(
        [refined_p.T, cid_p.T, score_p.T, jnp.zeros((2, _NP), f32)], axis=0)

    out = pl.pallas_call(
        _stage_b,
        out_shape=jax.ShapeDtypeStruct((8, 128), f32),
        compiler_params=pltpu.CompilerParams(
            vmem_limit_bytes=100 * 1024 * 1024),
        scratch_shapes=[
            pltpu.VMEM((_NP, _NP), jnp.int8),
            pltpu.VMEM((_NP, _NP), jnp.int8),
            pltpu.VMEM((1, _NP), f32),
            pltpu.VMEM((1, _NP), jnp.int8),
            pltpu.VMEM((_NP, 1), f32),
        ],
    )(nmsb_p, mscore_p, nmsb_p.T, mscore_p.T, packed)

    return out[:6, :].T[:_MAXDET, :]
